# Initial kernel scaffold; baseline (speedup 1.0000x reference)
import functools

import jax
import jax.numpy as jnp
from jax import lax
from jax.experimental import pallas as pl
from jax.experimental.pallas import tpu as pltpu
from jax.experimental.pallas import tpu_sc as plsc

_DIM = 32
_NC = 2
_NS = 16
_NW = _NC * _NS
_CHUNK = 1024


def _make_emb(batch: int):
  b_per_w = batch // _NW
  n_chunks = b_per_w // _CHUNK
  mesh = plsc.VectorSubcoreMesh(core_axis_name="c", subcore_axis_name="s")

  @functools.partial(
      pl.kernel,
      mesh=mesh,
      out_type=jax.ShapeDtypeStruct((batch, _DIM), jnp.float32),
      scratch_types=[
          pltpu.VMEM((_CHUNK,), jnp.int32),
          pltpu.VMEM((_CHUNK, _DIM), jnp.float32),
          pltpu.SemaphoreType.DMA,
      ],
  )
  def emb(idx_hbm, table_hbm, out_hbm, idx_v, rows_v, sem):
    wid = lax.axis_index("s") * _NC + lax.axis_index("c")
    base = wid * b_per_w

    def body(i, _):
      off = base + i * _CHUNK
      pltpu.sync_copy(idx_hbm.at[pl.ds(off, _CHUNK)], idx_v)
      pltpu.async_copy(table_hbm.at[idx_v], rows_v, sem).wait()
      pltpu.sync_copy(rows_v, out_hbm.at[pl.ds(off, _CHUNK)])
      return 0

    lax.fori_loop(0, n_chunks, body, 0)

  return emb


def kernel(x, weight):
  batch = x.size
  out = _make_emb(batch)(x.reshape(-1), weight)
  return out.reshape(x.shape + (_DIM,))


# speedup vs baseline: 1.5352x; 1.5352x over previous
import functools

import jax
import jax.numpy as jnp
from jax import lax
from jax.experimental import pallas as pl
from jax.experimental.pallas import tpu as pltpu
from jax.experimental.pallas import tpu_sc as plsc

_DIM = 32
_NC = 2
_NS = 16
_NW = _NC * _NS
_CHUNK = 1024


def _make_emb(batch: int):
  b_per_w = batch // _NW
  n_chunks = b_per_w // _CHUNK
  mesh = plsc.VectorSubcoreMesh(core_axis_name="c", subcore_axis_name="s")

  @functools.partial(
      pl.kernel,
      mesh=mesh,
      out_type=jax.ShapeDtypeStruct((batch, _DIM), jnp.float32),
      scratch_types=[
          pltpu.VMEM((_CHUNK,), jnp.int32),
          pltpu.VMEM((_CHUNK, _DIM), jnp.float32),
          pltpu.SemaphoreType.DMA,
      ],
      compiler_params=pltpu.CompilerParams(use_tc_tiling_on_sc=False),
  )
  def emb(idx_hbm, table_hbm, out_hbm, idx_v, rows_v, sem):
    wid = lax.axis_index("s") * _NC + lax.axis_index("c")
    base = wid * b_per_w

    def body(i, _):
      off = base + i * _CHUNK
      pltpu.sync_copy(idx_hbm.at[pl.ds(off, _CHUNK)], idx_v)
      pltpu.async_copy(table_hbm.at[idx_v], rows_v, sem).wait()
      pltpu.sync_copy(rows_v, out_hbm.at[pl.ds(off, _CHUNK)])
      return 0

    lax.fori_loop(0, n_chunks, body, 0)

  return emb


def kernel(x, weight):
  batch = x.size
  out = _make_emb(batch)(x.reshape(-1), weight)
  return out.reshape(x.shape + (_DIM,))


# speedup vs baseline: 1.5822x; 1.0307x over previous
import functools

import jax
import jax.numpy as jnp
from jax import lax
from jax.experimental import pallas as pl
from jax.experimental.pallas import tpu as pltpu
from jax.experimental.pallas import tpu_sc as plsc

_DIM = 32
_NC = 2
_NS = 16
_NW = _NC * _NS
_CHUNK = 800
_NBUF = 4


def _make_emb(batch: int):
  b_per_w = batch // _NW
  n_chunks = b_per_w // _CHUNK
  n_groups = n_chunks // _NBUF
  assert n_chunks % _NBUF == 0 and n_groups >= 2
  mesh = plsc.VectorSubcoreMesh(core_axis_name="c", subcore_axis_name="s")

  @functools.partial(
      pl.kernel,
      mesh=mesh,
      out_type=jax.ShapeDtypeStruct((batch, _DIM), jnp.float32),
      scratch_types=[
          pltpu.VMEM((b_per_w,), jnp.int32),
          pltpu.VMEM((_NBUF, _CHUNK, _DIM), jnp.float32),
          pltpu.SemaphoreType.DMA((_NBUF,)),
          pltpu.SemaphoreType.DMA((_NBUF,)),
      ],
      compiler_params=pltpu.CompilerParams(use_tc_tiling_on_sc=False),
  )
  def emb(idx_hbm, table_hbm, out_hbm, idx_full, bufs, gsems, wsems):
    wid = lax.axis_index("s") * _NC + lax.axis_index("c")
    base = wid * b_per_w
    pltpu.sync_copy(idx_hbm.at[pl.ds(base, b_per_w)], idx_full)

    def gather(i, b):
      return pltpu.make_async_copy(
          table_hbm.at[idx_full.at[pl.ds(i * _CHUNK, _CHUNK)]],
          bufs.at[b],
          gsems.at[b],
      )

    def wb(i, b):
      return pltpu.make_async_copy(
          bufs.at[b],
          out_hbm.at[pl.ds(base + i * _CHUNK, _CHUNK)],
          wsems.at[b],
      )

    def slot(i, b, do_a, do_b):
      if do_a:
        wb(i - 1, (b - 1) % _NBUF).wait()
      if do_b:
        gather(i + _NBUF - 1, (b - 1) % _NBUF).start()
      gather(i, b).wait()
      wb(i, b).start()

    for b in range(_NBUF):
      gather(b, b).start()

    slot(0, 0, False, False)
    for b in range(1, _NBUF):
      slot(b, b, True, True)

    def body(g, _):
      i0 = g * _NBUF
      for b in range(_NBUF):
        slot(i0 + b, b, True, True)
      return 0

    lax.fori_loop(1, n_groups - 1, body, 0)

    i0 = (n_groups - 1) * _NBUF
    slot(i0, 0, True, True)
    for b in range(1, _NBUF):
      slot(i0 + b, b, True, False)

    wb(n_chunks - 1, _NBUF - 1).wait()

  return emb


def kernel(x, weight):
  batch = x.size
  out = _make_emb(batch)(x.reshape(-1), weight)
  return out.reshape(x.shape + (_DIM,))
